# baseline (speedup 1.0000x reference)
import functools

import jax
import jax.numpy as jnp
from jax import lax
from jax.experimental import pallas as pl
from jax.experimental.pallas import tpu as pltpu
from jax.experimental.pallas import tpu_sc as plsc

VOCAB = 1000
EMBED = 64
SEQ = 200
BATCH = 4096
NC, NS, LANES = 2, 16, 16
NW = NC * NS
B_PER_W = BATCH // NW
NGRP = B_PER_W // LANES
TSTRIDE = EMBED + 1


def _sc_embed(idx_flat, table_flat):
    mesh = plsc.VectorSubcoreMesh(core_axis_name="c", subcore_axis_name="s")

    @functools.partial(
        pl.kernel,
        mesh=mesh,
        out_type=jax.ShapeDtypeStruct((SEQ, EMBED, BATCH), jnp.float32),
        scratch_types=[
            pltpu.VMEM((B_PER_W * SEQ,), jnp.int32),
            pltpu.VMEM((VOCAB * TSTRIDE,), jnp.float32),
            pltpu.VMEM((2, EMBED, B_PER_W), jnp.float32),
            pltpu.VMEM((2, EMBED, B_PER_W), jnp.float32),
            pltpu.SemaphoreType.DMA,
            pltpu.SemaphoreType.DMA,
        ],
        compiler_params=pltpu.CompilerParams(
            use_tc_tiling_on_sc=True, needs_layout_passes=False
        ),
    )
    def body(idx_hbm, tab_hbm, out_hbm, idx_v, tab_v, stage0, stage1,
             osem0, osem1):
        wid = lax.axis_index("s") * NC + lax.axis_index("c")
        b0 = wid * B_PER_W

        stages = (stage0, stage1)
        osems = (osem0, osem1)

        pltpu.sync_copy(idx_hbm.at[pl.ds(b0 * SEQ, B_PER_W * SEQ)], idx_v)
        pltpu.sync_copy(tab_hbm, tab_v)

        lane = lax.iota(jnp.int32, LANES)

        def compute_l(l, buf, j):
            for g in range(NGRP):
                iv = (lane + (g * LANES)) * SEQ + l
                idx16 = plsc.load_gather(idx_v, [iv])
                fidx = idx16 * TSTRIDE
                prev = None
                for e0 in range(0, EMBED, 8):
                    cur = []
                    for k in range(8):
                        cur.append(plsc.load_gather(tab_v, [fidx + (e0 + k)]))
                        if prev is not None:
                            stages[buf][j, e0 - 8 + k, pl.ds(g * LANES, LANES)] = prev[k]
                    prev = cur
                for k in range(8):
                    stages[buf][j, EMBED - 8 + k, pl.ds(g * LANES, LANES)] = prev[k]

        def fire_store(l, buf):
            pltpu.async_copy(
                stages[buf], out_hbm.at[pl.ds(l, 2), :, pl.ds(b0, B_PER_W)],
                osems[buf],
            )

        def wait_store(buf):
            pltpu.make_async_copy(
                stages[buf], out_hbm.at[pl.ds(0, 2), :, pl.ds(b0, B_PER_W)],
                osems[buf],
            ).wait()

        def quad(lq, carry):
            l = lq * 4

            @pl.when(lq > 0)
            def _():
                wait_store(0)

            compute_l(l, 0, 0)
            compute_l(l + 1, 0, 1)
            fire_store(l, 0)

            @pl.when(lq > 0)
            def _():
                wait_store(1)

            compute_l(l + 2, 1, 0)
            compute_l(l + 3, 1, 1)
            fire_store(l + 2, 1)
            return carry

        lax.fori_loop(0, SEQ // 4, quad, 0)
        wait_store(0)
        wait_store(1)

    return body(idx_flat, table_flat)


def kernel(words_seq, table):
    idx_flat = words_seq.astype(jnp.int32).reshape(-1)
    table_pad = jnp.pad(table.astype(jnp.float32), ((0, 0), (0, TSTRIDE - EMBED)))
    table_flat = table_pad.reshape(-1)
    out_t = _sc_embed(idx_flat, table_flat)
    return jnp.transpose(out_t, (2, 0, 1))


# speedup vs baseline: 1.8869x; 1.8869x over previous
import functools

import jax
import jax.numpy as jnp
from jax import lax
from jax.experimental import pallas as pl
from jax.experimental.pallas import tpu as pltpu
from jax.experimental.pallas import tpu_sc as plsc

VOCAB = 1000
EMBED = 64
SEQ = 200
BATCH = 4096
NC, NS, LANES = 2, 16, 16
NW = NC * NS
B_PER_W = BATCH // NW
NGRP = B_PER_W // LANES
TSTRIDE = EMBED + 1


def _sc_embed(idx_flat, table_flat):
    mesh = plsc.VectorSubcoreMesh(core_axis_name="c", subcore_axis_name="s")

    @functools.partial(
        pl.kernel,
        mesh=mesh,
        out_type=jax.ShapeDtypeStruct((SEQ, EMBED, BATCH), jnp.float32),
        scratch_types=[
            pltpu.VMEM((B_PER_W * SEQ,), jnp.int32),
            pltpu.VMEM((VOCAB * TSTRIDE,), jnp.float32),
            pltpu.VMEM((EMBED, B_PER_W), jnp.float32),
            pltpu.VMEM((EMBED, B_PER_W), jnp.float32),
            pltpu.SemaphoreType.DMA,
            pltpu.SemaphoreType.DMA,
        ],
        compiler_params=pltpu.CompilerParams(
            use_tc_tiling_on_sc=True, needs_layout_passes=False
        ),
    )
    def body(idx_hbm, tab_hbm, out_hbm, idx_v, tab_v, stage0, stage1,
             osem0, osem1):
        wid = lax.axis_index("s") * NC + lax.axis_index("c")
        b0 = wid * B_PER_W

        stages = (stage0, stage1)
        osems = (osem0, osem1)

        c1 = pltpu.async_copy(
            idx_hbm.at[pl.ds(b0 * SEQ, B_PER_W * SEQ)], idx_v, osem0
        )
        c2 = pltpu.async_copy(tab_hbm, tab_v, osem1)
        c1.wait()
        c2.wait()

        lane = lax.iota(jnp.int32, LANES)

        def compute_l(l, buf):
            for g in range(NGRP):
                iv = (lane + (g * LANES)) * SEQ + l
                idx16 = plsc.load_gather(idx_v, [iv])
                fidx = idx16 * TSTRIDE
                prev = None
                for e0 in range(0, EMBED, 8):
                    cur = []
                    for k in range(8):
                        cur.append(plsc.load_gather(tab_v, [fidx + (e0 + k)]))
                        if prev is not None:
                            stages[buf][e0 - 8 + k, pl.ds(g * LANES, LANES)] = prev[k]
                    prev = cur
                for k in range(8):
                    stages[buf][EMBED - 8 + k, pl.ds(g * LANES, LANES)] = prev[k]

        def fire_store(l, buf):
            pltpu.async_copy(
                stages[buf], out_hbm.at[l, :, pl.ds(b0, B_PER_W)], osems[buf]
            )

        def wait_store(buf):
            pltpu.make_async_copy(
                stages[buf], out_hbm.at[0, :, pl.ds(b0, B_PER_W)], osems[buf]
            ).wait()

        def pair(lp, carry):
            l = lp * 2

            @pl.when(lp > 0)
            def _():
                wait_store(0)

            compute_l(l, 0)
            fire_store(l, 0)

            @pl.when(lp > 0)
            def _():
                wait_store(1)

            compute_l(l + 1, 1)
            fire_store(l + 1, 1)
            return carry

        lax.fori_loop(0, SEQ // 2, pair, 0)
        wait_store(0)
        wait_store(1)

    return body(idx_flat, table_flat)


def kernel(words_seq, table):
    idx_flat = words_seq.astype(jnp.int32).reshape(-1)
    table_pad = jnp.pad(table.astype(jnp.float32), ((0, 0), (0, TSTRIDE - EMBED)))
    table_flat = table_pad.reshape(-1)
    out_t = _sc_embed(idx_flat, table_flat)
    return jnp.transpose(out_t, (2, 0, 1))
